# baseline (speedup 1.0000x reference)
import functools

import jax
import jax.numpy as jnp
from jax import lax
from jax.experimental import pallas as pl
from jax.experimental.pallas import tpu as pltpu
from jax.experimental.pallas import tpu_sc as plsc

_PREC = lax.Precision.DEFAULT
_QUANT_EVERY = 2

_SC_CORES = 2
_SC_SUBCORES = 16
_SC_WORKERS = _SC_CORES * _SC_SUBCORES


def _encode_body(x_ref, w_ref, b_ref, h_ref):
    h = lax.dot_general(x_ref[0].astype(jnp.bfloat16),
                        w_ref[...].astype(jnp.bfloat16),
                        (((0,), (0,)), ((), ())),
                        precision=_PREC, preferred_element_type=jnp.float32)
    h_ref[0] = h + b_ref[...]


def _dist_body(h_ref, cb_ref, idx_ref, *, kb, n_kb, tb, k_total):
    h = h_ref[...]
    hh = jnp.sum(h * h, axis=1)

    def step(k, carry):
        best_val, best_idx = carry
        off = pl.multiple_of(k * kb, kb)
        cb = cb_ref[pl.ds(off, kb), :]
        scores = lax.dot_general(h.astype(jnp.bfloat16), cb.astype(jnp.bfloat16),
                                 (((1,), (1,)), ((), ())),
                                 precision=_PREC,
                                 preferred_element_type=jnp.float32)
        cnorm = jnp.sum(cb * cb, axis=1)
        s = -(hh[:, None] - 2.0 * scores + cnorm[None, :])
        m = jnp.max(s, axis=1)
        col = lax.broadcasted_iota(jnp.int32, (tb, kb), 1)
        li = jnp.min(jnp.where(s == m[:, None], col, k_total), axis=1)
        li = li.astype(jnp.int32) + (k * kb).astype(jnp.int32)
        upd = m > best_val
        best_val = jnp.where(upd, m, best_val)
        best_idx = jnp.where(upd, li, best_idx)
        qnow = (k % _QUANT_EVERY) == (_QUANT_EVERY - 1)
        bq = best_val.astype(jnp.bfloat16).astype(jnp.float32)
        best_val = jnp.where(qnow, bq, best_val)
        return (best_val, best_idx)

    init = (jnp.full((tb,), -jnp.inf, jnp.float32),
            jnp.zeros((tb,), jnp.int32))
    _, best_idx = lax.fori_loop(0, n_kb, step, init)
    idx_ref[0, 0] = best_idx


def _decode_body(q_ref, w_ref, b_ref, o_ref):
    o = lax.dot_general(w_ref[...].astype(jnp.bfloat16),
                        q_ref[0].astype(jnp.bfloat16),
                        (((0,), (1,)), ((), ())),
                        precision=_PREC, preferred_element_type=jnp.float32)
    o_ref[0] = o + b_ref[...]


def _make_gather(bt, d, k_rows):
    rows_per_w = bt // _SC_WORKERS
    chunk = 128
    n_chunks = rows_per_w // chunk
    mesh = plsc.VectorSubcoreMesh(core_axis_name="c", subcore_axis_name="s")

    @functools.partial(
        pl.kernel,
        out_type=jax.ShapeDtypeStruct((bt, d), jnp.float32),
        mesh=mesh,
        scratch_types=(
            [pltpu.VMEM((chunk,), jnp.int32) for _ in range(n_chunks)]
            + [pltpu.VMEM((chunk, d), jnp.float32) for _ in range(n_chunks)]
            + [pltpu.SemaphoreType.DMA for _ in range(n_chunks)]
        ),
    )
    def gather(cb_hbm, idx_hbm, out_hbm, *scratch):
        idx_v = scratch[:n_chunks]
        rows_v = scratch[n_chunks:2 * n_chunks]
        sems = scratch[2 * n_chunks:]
        wid = lax.axis_index("s") * _SC_CORES + lax.axis_index("c")
        base = wid * rows_per_w
        copies = []
        for j in range(n_chunks):
            pltpu.sync_copy(idx_hbm.at[pl.ds(base + j * chunk, chunk)], idx_v[j])
            copies.append(
                pltpu.async_copy(cb_hbm.at[idx_v[j]], rows_v[j], sems[j]))
        for j in range(n_chunks):
            copies[j].wait()
            pltpu.sync_copy(rows_v[j], out_hbm.at[pl.ds(base + j * chunk, chunk)])

    return gather


def kernel(hidden_states, W_in, b_in, codebook, W_out, b_out):
    B, H, T = hidden_states.shape
    D = W_in.shape[1]
    K = codebook.shape[0]
    BT = B * T
    TB = 512
    KB = 1024
    G_T = BT // TB

    h = pl.pallas_call(
        _encode_body,
        grid=(B,),
        in_specs=[
            pl.BlockSpec((1, H, T), lambda b: (b, 0, 0)),
            pl.BlockSpec((H, D), lambda b: (0, 0)),
            pl.BlockSpec((1, D), lambda b: (0, 0)),
        ],
        out_specs=pl.BlockSpec((1, T, D), lambda b: (b, 0, 0)),
        out_shape=jax.ShapeDtypeStruct((B, T, D), jnp.float32),
    )(hidden_states, W_in, b_in.reshape(1, D))

    idx3 = pl.pallas_call(
        functools.partial(_dist_body, kb=KB, n_kb=K // KB, tb=TB, k_total=K),
        grid=(G_T,),
        in_specs=[
            pl.BlockSpec((TB, D), lambda i: (i, 0)),
            pl.BlockSpec((K, D), lambda i: (0, 0)),
        ],
        out_specs=pl.BlockSpec((1, 1, TB), lambda i: (i, 0, 0)),
        out_shape=jax.ShapeDtypeStruct((G_T, 1, TB), jnp.int32),
    )(h.reshape(BT, D), codebook)

    q = _make_gather(BT, D, K)(codebook, idx3.reshape(BT))

    out = pl.pallas_call(
        _decode_body,
        grid=(B,),
        in_specs=[
            pl.BlockSpec((1, T, D), lambda b: (b, 0, 0)),
            pl.BlockSpec((D, H), lambda b: (0, 0)),
            pl.BlockSpec((H, 1), lambda b: (0, 0)),
        ],
        out_specs=pl.BlockSpec((1, H, T), lambda b: (b, 0, 0)),
        out_shape=jax.ShapeDtypeStruct((B, H, T), jnp.float32),
    )(q.reshape(B, T, D), W_out, b_out.reshape(H, 1))

    return out


# speedup vs baseline: 1.0324x; 1.0324x over previous
import functools

import jax
import jax.numpy as jnp
from jax import lax
from jax.experimental import pallas as pl
from jax.experimental.pallas import tpu as pltpu
from jax.experimental.pallas import tpu_sc as plsc

_PREC = lax.Precision.DEFAULT
_QUANT_EVERY = 2

_SC_CORES = 2
_SC_SUBCORES = 16
_SC_WORKERS = _SC_CORES * _SC_SUBCORES


def _dist_body(x_ref, w_ref, b_ref, cb_ref, idx_ref, *, kb, n_kb, tb, k_total):
    h = lax.dot_general(x_ref[0].astype(jnp.bfloat16),
                        w_ref[...].astype(jnp.bfloat16),
                        (((0,), (0,)), ((), ())),
                        precision=_PREC, preferred_element_type=jnp.float32)
    h = h + b_ref[...]
    hh = jnp.sum(h * h, axis=1)

    def step(k, carry):
        best_val, best_idx = carry
        off = pl.multiple_of(k * kb, kb)
        cb = cb_ref[pl.ds(off, kb), :]
        scores = lax.dot_general(h.astype(jnp.bfloat16), cb.astype(jnp.bfloat16),
                                 (((1,), (1,)), ((), ())),
                                 precision=_PREC,
                                 preferred_element_type=jnp.float32)
        cnorm = jnp.sum(cb * cb, axis=1)
        s = -(hh[:, None] - 2.0 * scores + cnorm[None, :])
        m = jnp.max(s, axis=1)
        col = lax.broadcasted_iota(jnp.int32, (tb, kb), 1)
        li = jnp.min(jnp.where(s == m[:, None], col, k_total), axis=1)
        li = li.astype(jnp.int32) + (k * kb).astype(jnp.int32)
        upd = m > best_val
        best_val = jnp.where(upd, m, best_val)
        best_idx = jnp.where(upd, li, best_idx)
        qnow = (k % _QUANT_EVERY) == (_QUANT_EVERY - 1)
        bq = best_val.astype(jnp.bfloat16).astype(jnp.float32)
        best_val = jnp.where(qnow, bq, best_val)
        return (best_val, best_idx)

    init = (jnp.full((tb,), -jnp.inf, jnp.float32),
            jnp.zeros((tb,), jnp.int32))
    _, best_idx = lax.fori_loop(0, n_kb, step, init)
    idx_ref[0, 0] = best_idx


def _decode_body(q_ref, w_ref, b_ref, o_ref):
    o = lax.dot_general(w_ref[...].astype(jnp.bfloat16),
                        q_ref[0].astype(jnp.bfloat16),
                        (((0,), (1,)), ((), ())),
                        precision=_PREC, preferred_element_type=jnp.float32)
    o_ref[0] = o + b_ref[...]


def _make_gather(bt, d, k_rows):
    rows_per_w = bt // _SC_WORKERS
    chunk = 128
    n_chunks = rows_per_w // chunk
    mesh = plsc.VectorSubcoreMesh(core_axis_name="c", subcore_axis_name="s")

    @functools.partial(
        pl.kernel,
        out_type=jax.ShapeDtypeStruct((bt, d), jnp.float32),
        mesh=mesh,
        scratch_types=(
            [pltpu.VMEM((chunk,), jnp.int32) for _ in range(n_chunks)]
            + [pltpu.VMEM((chunk, d), jnp.float32) for _ in range(n_chunks)]
            + [pltpu.SemaphoreType.DMA for _ in range(n_chunks)]
        ),
    )
    def gather(cb_hbm, idx_hbm, out_hbm, *scratch):
        idx_v = scratch[:n_chunks]
        rows_v = scratch[n_chunks:2 * n_chunks]
        sems = scratch[2 * n_chunks:]
        wid = lax.axis_index("s") * _SC_CORES + lax.axis_index("c")
        base = wid * rows_per_w
        copies = []
        for j in range(n_chunks):
            pltpu.sync_copy(idx_hbm.at[pl.ds(base + j * chunk, chunk)], idx_v[j])
            copies.append(
                pltpu.async_copy(cb_hbm.at[idx_v[j]], rows_v[j], sems[j]))
        for j in range(n_chunks):
            copies[j].wait()
            pltpu.sync_copy(rows_v[j], out_hbm.at[pl.ds(base + j * chunk, chunk)])

    return gather


def kernel(hidden_states, W_in, b_in, codebook, W_out, b_out):
    B, H, T = hidden_states.shape
    D = W_in.shape[1]
    K = codebook.shape[0]
    BT = B * T
    TB = 512
    KB = 1024
    G_T = BT // TB

    tpb = T // TB

    idx3 = pl.pallas_call(
        functools.partial(_dist_body, kb=KB, n_kb=K // KB, tb=TB, k_total=K),
        grid=(G_T,),
        in_specs=[
            pl.BlockSpec((1, H, TB), lambda i: (i // tpb, 0, i % tpb)),
            pl.BlockSpec((H, D), lambda i: (0, 0)),
            pl.BlockSpec((1, D), lambda i: (0, 0)),
            pl.BlockSpec((K, D), lambda i: (0, 0)),
        ],
        out_specs=pl.BlockSpec((1, 1, TB), lambda i: (i, 0, 0)),
        out_shape=jax.ShapeDtypeStruct((G_T, 1, TB), jnp.int32),
    )(hidden_states, W_in, b_in.reshape(1, D), codebook)

    q = _make_gather(BT, D, K)(codebook, idx3.reshape(BT))

    out = pl.pallas_call(
        _decode_body,
        grid=(B,),
        in_specs=[
            pl.BlockSpec((1, T, D), lambda b: (b, 0, 0)),
            pl.BlockSpec((D, H), lambda b: (0, 0)),
            pl.BlockSpec((H, 1), lambda b: (0, 0)),
        ],
        out_specs=pl.BlockSpec((1, H, T), lambda b: (b, 0, 0)),
        out_shape=jax.ShapeDtypeStruct((B, H, T), jnp.float32),
    )(q.reshape(B, T, D), W_out, b_out.reshape(H, 1))

    return out


# speedup vs baseline: 1.1207x; 1.0855x over previous
import functools

import jax
import jax.numpy as jnp
from jax import lax
from jax.experimental import pallas as pl
from jax.experimental.pallas import tpu as pltpu
from jax.experimental.pallas import tpu_sc as plsc

_PREC = lax.Precision.DEFAULT
_QUANT_EVERY = 2

_SC_CORES = 2
_SC_SUBCORES = 16
_SC_WORKERS = _SC_CORES * _SC_SUBCORES


def _dist_body(x_ref, w_ref, b_ref, cb_ref, idx_ref, cn_ref, *,
               kb, n_kb, tb, k_total):

    @pl.when(pl.program_id(0) == 0)
    def _():
        cn_ref[...] = jnp.sum(cb_ref[...] * cb_ref[...], axis=1)

    h = lax.dot_general(x_ref[0].astype(jnp.bfloat16),
                        w_ref[...].astype(jnp.bfloat16),
                        (((0,), (0,)), ((), ())),
                        precision=_PREC, preferred_element_type=jnp.float32)
    h = h + b_ref[...]
    hh = jnp.sum(h * h, axis=1)

    def step(k, carry):
        best_val, best_idx = carry
        off = pl.multiple_of(k * kb, kb)
        cb = cb_ref[pl.ds(off, kb), :]
        scores = lax.dot_general(h.astype(jnp.bfloat16), cb.astype(jnp.bfloat16),
                                 (((1,), (1,)), ((), ())),
                                 precision=_PREC,
                                 preferred_element_type=jnp.float32)
        cnorm = cn_ref[pl.ds(off, kb)]
        u = (hh[:, None] - 2.0 * scores) + cnorm[None, :]
        um = jnp.min(u, axis=1)
        col = lax.broadcasted_iota(jnp.int32, (tb, kb), 1)
        li = jnp.min(jnp.where(u == um[:, None], col, k_total), axis=1)
        li = li.astype(jnp.int32) + (k * kb).astype(jnp.int32)
        upd = um < best_val
        best_val = jnp.where(upd, um, best_val)
        best_idx = jnp.where(upd, li, best_idx)
        qnow = (k % _QUANT_EVERY) == (_QUANT_EVERY - 1)
        bq = best_val.astype(jnp.bfloat16).astype(jnp.float32)
        best_val = jnp.where(qnow, bq, best_val)
        return (best_val, best_idx)

    init = (jnp.full((tb,), jnp.inf, jnp.float32),
            jnp.zeros((tb,), jnp.int32))
    _, best_idx = lax.fori_loop(0, n_kb, step, init)
    idx_ref[0, 0] = best_idx


def _decode_body(q_ref, w_ref, b_ref, o_ref):
    o = lax.dot_general(w_ref[...].astype(jnp.bfloat16),
                        q_ref[0].astype(jnp.bfloat16),
                        (((0,), (1,)), ((), ())),
                        precision=_PREC, preferred_element_type=jnp.float32)
    o_ref[0] = o + b_ref[...]


def _make_gather(bt, d, k_rows):
    rows_per_w = bt // _SC_WORKERS
    chunk = 128
    n_chunks = rows_per_w // chunk
    mesh = plsc.VectorSubcoreMesh(core_axis_name="c", subcore_axis_name="s")

    @functools.partial(
        pl.kernel,
        out_type=jax.ShapeDtypeStruct((bt, d), jnp.float32),
        mesh=mesh,
        scratch_types=(
            [pltpu.VMEM((chunk,), jnp.int32) for _ in range(n_chunks)]
            + [pltpu.VMEM((chunk, d), jnp.float32) for _ in range(n_chunks)]
            + [pltpu.SemaphoreType.DMA for _ in range(n_chunks)]
        ),
    )
    def gather(cb_hbm, idx_hbm, out_hbm, *scratch):
        idx_v = scratch[:n_chunks]
        rows_v = scratch[n_chunks:2 * n_chunks]
        sems = scratch[2 * n_chunks:]
        wid = lax.axis_index("s") * _SC_CORES + lax.axis_index("c")
        base = wid * rows_per_w
        copies = []
        for j in range(n_chunks):
            pltpu.sync_copy(idx_hbm.at[pl.ds(base + j * chunk, chunk)], idx_v[j])
            copies.append(
                pltpu.async_copy(cb_hbm.at[idx_v[j]], rows_v[j], sems[j]))
        for j in range(n_chunks):
            copies[j].wait()
            pltpu.sync_copy(rows_v[j], out_hbm.at[pl.ds(base + j * chunk, chunk)])

    return gather


def kernel(hidden_states, W_in, b_in, codebook, W_out, b_out):
    B, H, T = hidden_states.shape
    D = W_in.shape[1]
    K = codebook.shape[0]
    BT = B * T
    TB = 512
    KB = 1024
    G_T = BT // TB

    tpb = T // TB

    idx3 = pl.pallas_call(
        functools.partial(_dist_body, kb=KB, n_kb=K // KB, tb=TB, k_total=K),
        grid=(G_T,),
        in_specs=[
            pl.BlockSpec((1, H, TB), lambda i: (i // tpb, 0, i % tpb)),
            pl.BlockSpec((H, D), lambda i: (0, 0)),
            pl.BlockSpec((1, D), lambda i: (0, 0)),
            pl.BlockSpec((K, D), lambda i: (0, 0)),
        ],
        out_specs=pl.BlockSpec((1, 1, TB), lambda i: (i, 0, 0)),
        out_shape=jax.ShapeDtypeStruct((G_T, 1, TB), jnp.int32),
        scratch_shapes=[pltpu.VMEM((K,), jnp.float32)],
    )(hidden_states, W_in, b_in.reshape(1, D), codebook)

    q = _make_gather(BT, D, K)(codebook, idx3.reshape(BT))

    out = pl.pallas_call(
        _decode_body,
        grid=(B,),
        in_specs=[
            pl.BlockSpec((1, T, D), lambda b: (b, 0, 0)),
            pl.BlockSpec((D, H), lambda b: (0, 0)),
            pl.BlockSpec((H, 1), lambda b: (0, 0)),
        ],
        out_specs=pl.BlockSpec((1, H, T), lambda b: (b, 0, 0)),
        out_shape=jax.ShapeDtypeStruct((B, H, T), jnp.float32),
    )(q.reshape(B, T, D), W_out, b_out.reshape(H, 1))

    return out
